# baseline (speedup 1.0000x reference)
import jax
import jax.numpy as jnp
from jax import lax
from jax.experimental import pallas as pl
from jax.experimental.pallas import tpu as pltpu
from jax.experimental.pallas import tpu_sc as plsc

N = 10000
E = 320000
D = 128
H = 128

NC = 2
NS = 16
NW = NC * NS

K = 128
C0 = 153
C1 = 5
EW0 = C0 * K
EW1 = C1 * K
E_PAD = NS * (EW0 + EW1)
RPS = 632
NP = NS * RPS


def _sc_aggregate(with_deg: bool):
    agg_t = jax.ShapeDtypeStruct((NC, NP, D), jnp.float32)
    if with_deg:
        out_type = [agg_t, jax.ShapeDtypeStruct((NC * NP,), jnp.float32)]
    else:
        out_type = agg_t

    scratch = [
        pltpu.VMEM_SHARED((NP, D), jnp.float32),
        pltpu.VMEM_SHARED((NP,), jnp.float32),
        pltpu.VMEM((2, K), jnp.int32),
        pltpu.VMEM((2, K), jnp.int32),
        pltpu.VMEM((K, D), jnp.float32),
        pltpu.VMEM((K, D), jnp.float32),
        pltpu.VMEM((K,), jnp.float32),
        pltpu.VMEM((RPS,), jnp.float32),
        pltpu.SemaphoreType.DMA,
        pltpu.SemaphoreType.DMA,
        pltpu.SemaphoreType.DMA,
        pltpu.SemaphoreType.DMA,
        pltpu.SemaphoreType.DMA,
        pltpu.SemaphoreType.DMA,
    ]

    def body(srcf, dstf, p, *rest):
        if with_deg:
            agg_out, deg_out = rest[0], rest[1]
            rest = rest[2:]
        else:
            agg_out, deg_out = rest[0], None
            rest = rest[1:]
        (acc_sh, deg_sh, srcb, dstb, rows0, rows1, ones_v, deg_v,
         sem0, sem1, semis0, semis1, semid0, semid1) = rest
        semis = (semis0, semis1)
        semid = (semid0, semid1)
        rows = (rows0, rows1)
        semr = (sem0, sem1)

        cid = lax.axis_index("c")
        sid = lax.axis_index("s")
        r0 = sid * RPS
        is0 = cid == 0
        base = jnp.where(is0, sid * EW0, NS * EW0 + sid * EW1)
        T = jnp.where(is0, (C0 - 3) // 2, (C1 - 3) // 2)

        def issue_idx(j, slot):
            off = base + j * K
            pltpu.async_copy(srcf.at[pl.ds(off, K)], srcb.at[slot], semis[slot])
            pltpu.async_copy(dstf.at[pl.ds(off, K)], dstb.at[slot], semid[slot])

        def wait_src(slot):
            pltpu.make_async_copy(
                srcf.at[pl.ds(base, K)], srcb.at[slot], semis[slot]).wait()

        def wait_dst(slot):
            pltpu.make_async_copy(
                dstf.at[pl.ds(base, K)], dstb.at[slot], semid[slot]).wait()

        def gather(slot):
            pltpu.async_copy(p.at[srcb.at[slot]], rows[slot], semr[slot])

        def wait_rows(slot):
            pltpu.make_async_copy(
                p.at[srcb.at[slot]], rows[slot], semr[slot]).wait()

        def zrow(i, carry):
            for c in range(D // 16):
                rows0[i, pl.ds(c * 16, 16)] = jnp.zeros((16,), jnp.float32)
            return carry
        lax.fori_loop(0, K, zrow, 0)
        full, tail = RPS // K, RPS % K
        for t in range(full):
            pltpu.sync_copy(rows0, acc_sh.at[pl.ds(r0 + t * K, K)])
        if tail:
            pltpu.sync_copy(rows0.at[pl.ds(0, tail)],
                            acc_sh.at[pl.ds(r0 + full * K, tail)])
        if with_deg:
            def zstep(i, carry):
                deg_v[pl.ds(i * 16, 16)] = jnp.zeros((16,), jnp.float32)
                return carry
            lax.fori_loop(0, RPS // 16, zstep, 0)
            deg_v[pl.ds(RPS - 16, 16)] = jnp.zeros((16,), jnp.float32)
            pltpu.sync_copy(deg_v, deg_sh.at[pl.ds(r0, RPS)])
            for i in range(K // 16):
                ones_v[pl.ds(i * 16, 16)] = jnp.ones((16,), jnp.float32)
        plsc.subcore_barrier()

        def half(jc, cur, nxt, issue, gather_next):
            if gather_next:
                wait_src(nxt)
                gather(nxt)
            wait_rows(cur)
            wait_dst(cur)
            pltpu.sync_copy(rows[cur], acc_sh.at[dstb.at[cur]], add=True)
            if with_deg:
                pltpu.sync_copy(ones_v, deg_sh.at[dstb.at[cur]], add=True)
            if issue:
                issue_idx(jc + 2, cur)

        issue_idx(0, 0)
        issue_idx(1, 1)
        wait_src(0)
        gather(0)

        def step(i, carry):
            j = 2 * i
            half(j, 0, 1, True, True)
            half(j + 1, 1, 0, True, True)
            return carry

        lax.fori_loop(0, T, step, 0)
        half(2 * T, 0, 1, True, True)
        half(2 * T + 1, 1, 0, False, True)
        half(2 * T + 2, 0, 1, False, False)
        plsc.subcore_barrier()

        wfull, wtail = RPS // K, RPS % K
        pltpu.sync_copy(acc_sh.at[pl.ds(r0, K)], rows0)
        pltpu.async_copy(rows0, agg_out.at[cid, pl.ds(r0, K)], sem0)
        for t in range(1, wfull):
            cur = rows[t % 2]
            pltpu.sync_copy(acc_sh.at[pl.ds(r0 + t * K, K)], cur)
            pltpu.async_copy(cur, agg_out.at[cid, pl.ds(r0 + t * K, K)],
                             semr[t % 2])
            prev = rows[(t - 1) % 2]
            pltpu.make_async_copy(prev, agg_out.at[cid, pl.ds(r0, K)],
                                  semr[(t - 1) % 2]).wait()
        if wtail:
            tcur = rows[wfull % 2]
            pltpu.sync_copy(acc_sh.at[pl.ds(r0 + wfull * K, wtail)],
                            tcur.at[pl.ds(0, wtail)])
            pltpu.async_copy(tcur.at[pl.ds(0, wtail)],
                             agg_out.at[cid, pl.ds(r0 + wfull * K, wtail)],
                             semr[wfull % 2])
            pltpu.make_async_copy(
                rows[(wfull - 1) % 2], agg_out.at[cid, pl.ds(r0, K)],
                semr[(wfull - 1) % 2]).wait()
            pltpu.make_async_copy(
                tcur.at[pl.ds(0, wtail)],
                agg_out.at[cid, pl.ds(r0 + wfull * K, wtail)],
                semr[wfull % 2]).wait()
        else:
            pltpu.make_async_copy(
                rows[(wfull - 1) % 2], agg_out.at[cid, pl.ds(r0, K)],
                semr[(wfull - 1) % 2]).wait()
        if with_deg:
            pltpu.sync_copy(deg_sh.at[pl.ds(r0, RPS)], deg_v)
            pltpu.sync_copy(deg_v, deg_out.at[pl.ds(cid * NP + r0, RPS)])

    return pl.kernel(
        body,
        out_type=out_type,
        mesh=plsc.VectorSubcoreMesh(core_axis_name="c", subcore_axis_name="s"),
        scratch_types=scratch,
    )


_sc_agg_deg = _sc_aggregate(with_deg=True)
_sc_agg = _sc_aggregate(with_deg=False)


BN = 1000
GRID = (N // BN,)


def _tc1_body(x_ref, w_ref, b_ref, s_ref, p_ref):
    y = jnp.dot(x_ref[...], w_ref[...], preferred_element_type=jnp.float32)
    s_ref[...] = y[:, :H] + b_ref[...]
    p_ref[...] = y[:, H:]


def _tc2_body(s0_ref, a0_ref, a1_ref, d0_ref, d1_ref, w_ref, b_ref,
              s_ref, p_ref):
    deg = jnp.squeeze(d0_ref[...] + d1_ref[...], axis=0)
    inv = 1.0 / jnp.maximum(deg, 1.0)
    agg = jnp.squeeze(a0_ref[...] + a1_ref[...], axis=0)
    h = jnp.maximum(s0_ref[...] + agg * inv, 0.0)
    y = jnp.dot(h, w_ref[...], preferred_element_type=jnp.float32)
    s_ref[...] = y[:, :H] + b_ref[...]
    p_ref[...] = y[:, H:]


def _tc3_body(s1_ref, a0_ref, a1_ref, d0_ref, d1_ref, o_ref):
    deg = jnp.squeeze(d0_ref[...] + d1_ref[...], axis=0)
    inv = 1.0 / jnp.maximum(deg, 1.0)
    agg = jnp.squeeze(a0_ref[...] + a1_ref[...], axis=0)
    o_ref[...] = s1_ref[...] + agg * inv


def _row_spec():
    return pl.BlockSpec((BN, D), lambda i: (i, 0))


def _w_spec():
    return pl.BlockSpec((D, 2 * H), lambda i: (0, 0))


def _b_spec():
    return pl.BlockSpec((1, H), lambda i: (0, 0))


def _agg_spec(c):
    return pl.BlockSpec((1, BN, H), lambda i, c=c: (c, i, 0))


def _deg_spec(c):
    return pl.BlockSpec((1, BN, 1), lambda i, c=c: (c, i, 0))


_tc1 = pl.pallas_call(
    _tc1_body,
    grid=GRID,
    in_specs=[_row_spec(), _w_spec(), _b_spec()],
    out_specs=[_row_spec(), _row_spec()],
    out_shape=[jax.ShapeDtypeStruct((N, H), jnp.float32)] * 2,
)

_tc2 = pl.pallas_call(
    _tc2_body,
    grid=GRID,
    in_specs=[_row_spec(), _agg_spec(0), _agg_spec(1), _deg_spec(0),
              _deg_spec(1), _w_spec(), _b_spec()],
    out_specs=[_row_spec(), _row_spec()],
    out_shape=[jax.ShapeDtypeStruct((N, H), jnp.float32)] * 2,
)

_tc3 = pl.pallas_call(
    _tc3_body,
    grid=GRID,
    in_specs=[_row_spec(), _agg_spec(0), _agg_spec(1), _deg_spec(0),
              _deg_spec(1)],
    out_specs=_row_spec(),
    out_shape=jax.ShapeDtypeStruct((N, H), jnp.float32),
)


def kernel(x, edge_index, W_self0, W_neigh0, b0, W_self1, W_neigh1, b1):
    pad = E_PAD - E
    src3 = jnp.concatenate([edge_index[0], jnp.zeros((pad,), jnp.int32)])
    dst3 = jnp.concatenate([edge_index[1], jnp.full((pad,), N, jnp.int32)])
    w0 = jnp.concatenate([W_self0, W_neigh0], axis=1)
    w1 = jnp.concatenate([W_self1, W_neigh1], axis=1)
    s0, p0 = _tc1(x, w0, b0.reshape(1, H))
    agg0, deg0 = _sc_agg_deg(src3, dst3, p0)
    deg3 = deg0.reshape(NC, NP, 1)
    s1, p1 = _tc2(s0, agg0, agg0, deg3, deg3, w1, b1.reshape(1, H))
    agg1 = _sc_agg(src3, dst3, p1)
    out = _tc3(s1, agg1, agg1, deg3, deg3)
    return out


# speedup vs baseline: 1.0044x; 1.0044x over previous
import jax
import jax.numpy as jnp
from jax import lax
from jax.experimental import pallas as pl
from jax.experimental.pallas import tpu as pltpu
from jax.experimental.pallas import tpu_sc as plsc

N = 10000
E = 320000
D = 128
H = 128

NC = 2
NS = 16
NW = NC * NS

K = 128
C0 = 111
C1 = 47
EW0 = C0 * K
EW1 = C1 * K
E_PAD = NS * (EW0 + EW1)
RPS = 640
NP = NS * RPS


def _sc_aggregate(with_deg: bool):
    agg_t = jax.ShapeDtypeStruct((NC, NP, D), jnp.float32)
    if with_deg:
        out_type = [agg_t, jax.ShapeDtypeStruct((NC * NP,), jnp.float32)]
    else:
        out_type = agg_t

    scratch = [
        pltpu.VMEM_SHARED((NP, D), jnp.float32),
        pltpu.VMEM_SHARED((NP,), jnp.float32),
        pltpu.VMEM((2, K), jnp.int32),
        pltpu.VMEM((2, K), jnp.int32),
        pltpu.VMEM((K, D), jnp.float32),
        pltpu.VMEM((K, D), jnp.float32),
        pltpu.VMEM((K,), jnp.float32),
        pltpu.VMEM((RPS,), jnp.float32),
    ] + [pltpu.SemaphoreType.DMA] * 6

    def body(srcf, dstf, p, *rest):
        if with_deg:
            agg_out, deg_out = rest[0], rest[1]
            rest = rest[2:]
        else:
            agg_out, deg_out = rest[0], None
            rest = rest[1:]
        acc_sh, deg_sh, srcb, dstb = rest[0], rest[1], rest[2], rest[3]
        rows = rest[4:6]
        ones_v, deg_v = rest[6], rest[7]
        semr = rest[8:10]
        semis = rest[10:12]
        semid = rest[12:14]

        cid = lax.axis_index("c")
        sid = lax.axis_index("s")
        r0 = sid * RPS
        is0 = cid == 0
        base = jnp.where(is0, sid * EW0, NS * EW0 + sid * EW1)
        T = jnp.where(is0, (C0 - 3) // 2, (C1 - 3) // 2)

        def issue_idx(j, slot):
            off = base + j * K
            pltpu.async_copy(srcf.at[pl.ds(off, K)], srcb.at[slot],
                             semis[slot])
            pltpu.async_copy(dstf.at[pl.ds(off, K)], dstb.at[slot],
                             semid[slot])

        def wait_src(slot):
            pltpu.make_async_copy(
                srcf.at[pl.ds(base, K)], srcb.at[slot], semis[slot]).wait()

        def wait_dst(slot):
            pltpu.make_async_copy(
                dstf.at[pl.ds(base, K)], dstb.at[slot], semid[slot]).wait()

        def gather(slot):
            pltpu.async_copy(p.at[srcb.at[slot]], rows[slot], semr[slot])

        def wait_rows(slot):
            pltpu.make_async_copy(
                p.at[srcb.at[slot]], rows[slot], semr[slot]).wait()

        def scatter(slot):
            pltpu.sync_copy(rows[slot], acc_sh.at[dstb.at[slot]], add=True)
            if with_deg:
                pltpu.sync_copy(ones_v, deg_sh.at[dstb.at[slot]], add=True)

        def _work():
            def zrow(i, carry):
                for c in range(D // 16):
                    rows[0][i, pl.ds(c * 16, 16)] = jnp.zeros(
                        (16,), jnp.float32)
                return carry
            lax.fori_loop(0, K, zrow, 0)
            for t in range(RPS // K):
                pltpu.sync_copy(rows[0], acc_sh.at[pl.ds(r0 + t * K, K)])
            if with_deg:
                def zstep(i, carry):
                    deg_v[pl.ds(i * 16, 16)] = jnp.zeros((16,), jnp.float32)
                    return carry
                lax.fori_loop(0, RPS // 16, zstep, 0)
                pltpu.sync_copy(deg_v, deg_sh.at[pl.ds(r0, RPS)])
                for i in range(K // 16):
                    ones_v[pl.ds(i * 16, 16)] = jnp.ones((16,), jnp.float32)
            plsc.subcore_barrier()

            def half(jc, cur, nxt, issue, gather_next):
                if gather_next:
                    wait_src(nxt)
                    gather(nxt)
                wait_rows(cur)
                wait_dst(cur)
                scatter(cur)
                if issue:
                    issue_idx(jc + 2, cur)

            issue_idx(0, 0)
            issue_idx(1, 1)
            wait_src(0)
            gather(0)

            def step(i, carry):
                j = 2 * i
                half(j, 0, 1, True, True)
                half(j + 1, 1, 0, True, True)
                return carry

            lax.fori_loop(0, T, step, 0)
            half(2 * T, 0, 1, True, True)
            half(2 * T + 1, 1, 0, False, True)
            half(2 * T + 2, 0, 1, False, False)
            plsc.subcore_barrier()

            wfull = RPS // K
            pltpu.sync_copy(acc_sh.at[pl.ds(r0, K)], rows[0])
            pltpu.async_copy(rows[0], agg_out.at[cid, pl.ds(r0, K)], semr[0])
            for t in range(1, wfull):
                cur = rows[t % 2]
                pltpu.sync_copy(acc_sh.at[pl.ds(r0 + t * K, K)], cur)
                pltpu.async_copy(cur, agg_out.at[cid, pl.ds(r0 + t * K, K)],
                                 semr[t % 2])
                prev = rows[(t - 1) % 2]
                pltpu.make_async_copy(prev, agg_out.at[cid, pl.ds(r0, K)],
                                      semr[(t - 1) % 2]).wait()
            pltpu.make_async_copy(
                rows[(wfull - 1) % 2], agg_out.at[cid, pl.ds(r0, K)],
                semr[(wfull - 1) % 2]).wait()
            if with_deg:
                pltpu.sync_copy(deg_sh.at[pl.ds(r0, RPS)], deg_v)
                pltpu.sync_copy(deg_v,
                                deg_out.at[pl.ds(cid * NP + r0, RPS)])

        _work()

    return pl.kernel(
        body,
        out_type=out_type,
        mesh=plsc.VectorSubcoreMesh(core_axis_name="c", subcore_axis_name="s"),
        scratch_types=scratch,
    )


_sc_agg_deg = _sc_aggregate(with_deg=True)
_sc_agg = _sc_aggregate(with_deg=False)


BN = 1000
GRID = (N // BN,)


def _tc1_body(x_ref, w_ref, b_ref, s_ref, p_ref):
    y = jnp.dot(x_ref[...], w_ref[...], preferred_element_type=jnp.float32)
    s_ref[...] = y[:, :H] + b_ref[...]
    p_ref[...] = y[:, H:]


def _tc2_body(s0_ref, a0_ref, a1_ref, d0_ref, d1_ref, w_ref, b_ref,
              s_ref, p_ref):
    deg = jnp.squeeze(d0_ref[...] + d1_ref[...], axis=0)
    inv = 1.0 / jnp.maximum(deg, 1.0)
    agg = jnp.squeeze(a0_ref[...] + a1_ref[...], axis=0)
    h = jnp.maximum(s0_ref[...] + agg * inv, 0.0)
    y = jnp.dot(h, w_ref[...], preferred_element_type=jnp.float32)
    s_ref[...] = y[:, :H] + b_ref[...]
    p_ref[...] = y[:, H:]


def _tc3_body(s1_ref, a0_ref, a1_ref, d0_ref, d1_ref, o_ref):
    deg = jnp.squeeze(d0_ref[...] + d1_ref[...], axis=0)
    inv = 1.0 / jnp.maximum(deg, 1.0)
    agg = jnp.squeeze(a0_ref[...] + a1_ref[...], axis=0)
    o_ref[...] = s1_ref[...] + agg * inv


def _row_spec():
    return pl.BlockSpec((BN, D), lambda i: (i, 0))


def _w_spec():
    return pl.BlockSpec((D, 2 * H), lambda i: (0, 0))


def _b_spec():
    return pl.BlockSpec((1, H), lambda i: (0, 0))


def _agg_spec(c):
    return pl.BlockSpec((1, BN, H), lambda i, c=c: (c, i, 0))


def _deg_spec(c):
    return pl.BlockSpec((1, BN, 1), lambda i, c=c: (c, i, 0))


_tc1 = pl.pallas_call(
    _tc1_body,
    grid=GRID,
    in_specs=[_row_spec(), _w_spec(), _b_spec()],
    out_specs=[_row_spec(), _row_spec()],
    out_shape=[jax.ShapeDtypeStruct((N, H), jnp.float32)] * 2,
)

_tc2 = pl.pallas_call(
    _tc2_body,
    grid=GRID,
    in_specs=[_row_spec(), _agg_spec(0), _agg_spec(1), _deg_spec(0),
              _deg_spec(1), _w_spec(), _b_spec()],
    out_specs=[_row_spec(), _row_spec()],
    out_shape=[jax.ShapeDtypeStruct((N, H), jnp.float32)] * 2,
)

_tc3 = pl.pallas_call(
    _tc3_body,
    grid=GRID,
    in_specs=[_row_spec(), _agg_spec(0), _agg_spec(1), _deg_spec(0),
              _deg_spec(1)],
    out_specs=_row_spec(),
    out_shape=jax.ShapeDtypeStruct((N, H), jnp.float32),
)


def kernel(x, edge_index, W_self0, W_neigh0, b0, W_self1, W_neigh1, b1):
    pad = E_PAD - E
    src3 = jnp.concatenate([edge_index[0], jnp.zeros((pad,), jnp.int32)])
    dst3 = jnp.concatenate([edge_index[1], jnp.full((pad,), N, jnp.int32)])
    w0 = jnp.concatenate([W_self0, W_neigh0], axis=1)
    w1 = jnp.concatenate([W_self1, W_neigh1], axis=1)

    s0, p0 = _tc1(x, w0, b0.reshape(1, H))
    agg0, deg0 = _sc_agg_deg(src3, dst3, p0)
    deg3 = deg0.reshape(NC, NP, 1)
    s1, p1 = _tc2(s0, agg0, agg0, deg3, deg3, w1, b1.reshape(1, H))
    agg1 = _sc_agg(src3, dst3, p1)
    out = _tc3(s1, agg1, agg1, deg3, deg3)
    return out


# speedup vs baseline: 1.1105x; 1.1057x over previous
import jax
import jax.numpy as jnp
from jax import lax
from jax.experimental import pallas as pl
from jax.experimental.pallas import tpu as pltpu
from jax.experimental.pallas import tpu_sc as plsc

N = 10000
E = 320000
D = 128
H = 128

NC = 2
NS = 16
NW = NC * NS

K = 128
C0 = 139
C1 = 19
EW0 = C0 * K
EW1 = C1 * K
E_PAD = NS * (EW0 + EW1)
RPS = 640
NP = NS * RPS


def _sc_aggregate(with_deg: bool):
    agg_t = jax.ShapeDtypeStruct((NC, NP, D), jnp.float32)
    if with_deg:
        out_type = [agg_t, jax.ShapeDtypeStruct((NC * NP,), jnp.float32)]
    else:
        out_type = agg_t

    scratch = [
        pltpu.VMEM_SHARED((NP, D), jnp.float32),
        pltpu.VMEM_SHARED((NP,), jnp.float32),
        pltpu.VMEM((2, K), jnp.int32),
        pltpu.VMEM((2, K), jnp.int32),
        pltpu.VMEM((K, D), jnp.float32),
        pltpu.VMEM((K, D), jnp.float32),
        pltpu.VMEM((K,), jnp.float32),
        pltpu.VMEM((RPS,), jnp.float32),
    ] + [pltpu.SemaphoreType.DMA] * 6

    def body(srcf, dstf, p, *rest):
        if with_deg:
            agg_out, deg_out = rest[0], rest[1]
            rest = rest[2:]
        else:
            agg_out, deg_out = rest[0], None
            rest = rest[1:]
        acc_sh, deg_sh, srcb, dstb = rest[0], rest[1], rest[2], rest[3]
        rows = rest[4:6]
        ones_v, deg_v = rest[6], rest[7]
        semr = rest[8:10]
        semis = rest[10:12]
        semid = rest[12:14]

        cid = lax.axis_index("c")
        sid = lax.axis_index("s")
        r0 = sid * RPS
        is0 = cid == 0
        base = jnp.where(is0, sid * EW0, NS * EW0 + sid * EW1)
        T = jnp.where(is0, (C0 - 3) // 2, (C1 - 3) // 2)

        def issue_idx(j, slot):
            off = base + j * K
            pltpu.async_copy(srcf.at[pl.ds(off, K)], srcb.at[slot],
                             semis[slot])
            pltpu.async_copy(dstf.at[pl.ds(off, K)], dstb.at[slot],
                             semid[slot])

        def wait_src(slot):
            pltpu.make_async_copy(
                srcf.at[pl.ds(base, K)], srcb.at[slot], semis[slot]).wait()

        def wait_dst(slot):
            pltpu.make_async_copy(
                dstf.at[pl.ds(base, K)], dstb.at[slot], semid[slot]).wait()

        def gather(slot):
            pltpu.async_copy(p.at[srcb.at[slot]], rows[slot], semr[slot])

        def wait_rows(slot):
            pltpu.make_async_copy(
                p.at[srcb.at[slot]], rows[slot], semr[slot]).wait()

        def scatter(slot):
            pltpu.sync_copy(rows[slot], acc_sh.at[dstb.at[slot]], add=True)
            if with_deg:
                pltpu.sync_copy(ones_v, deg_sh.at[dstb.at[slot]], add=True)

        def _work():
            def zrow(i, carry):
                for c in range(D // 16):
                    rows[0][i, pl.ds(c * 16, 16)] = jnp.zeros(
                        (16,), jnp.float32)
                return carry
            lax.fori_loop(0, K, zrow, 0)
            for t in range(RPS // K):
                pltpu.sync_copy(rows[0], acc_sh.at[pl.ds(r0 + t * K, K)])
            if with_deg:
                def zstep(i, carry):
                    deg_v[pl.ds(i * 16, 16)] = jnp.zeros((16,), jnp.float32)
                    return carry
                lax.fori_loop(0, RPS // 16, zstep, 0)
                pltpu.sync_copy(deg_v, deg_sh.at[pl.ds(r0, RPS)])
                for i in range(K // 16):
                    ones_v[pl.ds(i * 16, 16)] = jnp.ones((16,), jnp.float32)
            plsc.subcore_barrier()

            def half(jc, cur, nxt, issue, gather_next):
                if gather_next:
                    wait_src(nxt)
                    gather(nxt)
                wait_rows(cur)
                wait_dst(cur)
                scatter(cur)
                if issue:
                    issue_idx(jc + 2, cur)

            issue_idx(0, 0)
            issue_idx(1, 1)
            wait_src(0)
            gather(0)

            def step(i, carry):
                j = 2 * i
                half(j, 0, 1, True, True)
                half(j + 1, 1, 0, True, True)
                return carry

            lax.fori_loop(0, T, step, 0)
            half(2 * T, 0, 1, True, True)
            half(2 * T + 1, 1, 0, False, True)
            half(2 * T + 2, 0, 1, False, False)
            plsc.subcore_barrier()

            wfull = RPS // K
            pltpu.sync_copy(acc_sh.at[pl.ds(r0, K)], rows[0])
            pltpu.async_copy(rows[0], agg_out.at[cid, pl.ds(r0, K)], semr[0])
            for t in range(1, wfull):
                cur = rows[t % 2]
                pltpu.sync_copy(acc_sh.at[pl.ds(r0 + t * K, K)], cur)
                pltpu.async_copy(cur, agg_out.at[cid, pl.ds(r0 + t * K, K)],
                                 semr[t % 2])
                prev = rows[(t - 1) % 2]
                pltpu.make_async_copy(prev, agg_out.at[cid, pl.ds(r0, K)],
                                      semr[(t - 1) % 2]).wait()
            pltpu.make_async_copy(
                rows[(wfull - 1) % 2], agg_out.at[cid, pl.ds(r0, K)],
                semr[(wfull - 1) % 2]).wait()
            if with_deg:
                pltpu.sync_copy(deg_sh.at[pl.ds(r0, RPS)], deg_v)
                pltpu.sync_copy(deg_v,
                                deg_out.at[pl.ds(cid * NP + r0, RPS)])

        _work()

    return pl.kernel(
        body,
        out_type=out_type,
        mesh=plsc.VectorSubcoreMesh(core_axis_name="c", subcore_axis_name="s"),
        scratch_types=scratch,
    )


_sc_agg_deg = _sc_aggregate(with_deg=True)
_sc_agg = _sc_aggregate(with_deg=False)


BN = 1000
GRID = (N // BN,)


def _tc1_body(x_ref, w_ref, b_ref, s_ref, p_ref):
    y = jnp.dot(x_ref[...], w_ref[...], preferred_element_type=jnp.float32)
    s_ref[...] = y[:, :H] + b_ref[...]
    p_ref[...] = y[:, H:]


def _tc2_body(s0_ref, a0_ref, a1_ref, d0_ref, d1_ref, w_ref, b_ref,
              s_ref, p_ref):
    deg = jnp.squeeze(d0_ref[...] + d1_ref[...], axis=0)
    inv = 1.0 / jnp.maximum(deg, 1.0)
    agg = jnp.squeeze(a0_ref[...] + a1_ref[...], axis=0)
    h = jnp.maximum(s0_ref[...] + agg * inv, 0.0)
    y = jnp.dot(h, w_ref[...], preferred_element_type=jnp.float32)
    s_ref[...] = y[:, :H] + b_ref[...]
    p_ref[...] = y[:, H:]


def _tc3_body(s1_ref, a0_ref, a1_ref, d0_ref, d1_ref, o_ref):
    deg = jnp.squeeze(d0_ref[...] + d1_ref[...], axis=0)
    inv = 1.0 / jnp.maximum(deg, 1.0)
    agg = jnp.squeeze(a0_ref[...] + a1_ref[...], axis=0)
    o_ref[...] = s1_ref[...] + agg * inv


def _row_spec():
    return pl.BlockSpec((BN, D), lambda i: (i, 0))


def _w_spec():
    return pl.BlockSpec((D, 2 * H), lambda i: (0, 0))


def _b_spec():
    return pl.BlockSpec((1, H), lambda i: (0, 0))


def _agg_spec(c):
    return pl.BlockSpec((1, BN, H), lambda i, c=c: (c, i, 0))


def _deg_spec(c):
    return pl.BlockSpec((1, BN, 1), lambda i, c=c: (c, i, 0))


_tc1 = pl.pallas_call(
    _tc1_body,
    grid=GRID,
    in_specs=[_row_spec(), _w_spec(), _b_spec()],
    out_specs=[_row_spec(), _row_spec()],
    out_shape=[jax.ShapeDtypeStruct((N, H), jnp.float32)] * 2,
)

_tc2 = pl.pallas_call(
    _tc2_body,
    grid=GRID,
    in_specs=[_row_spec(), _agg_spec(0), _agg_spec(1), _deg_spec(0),
              _deg_spec(1), _w_spec(), _b_spec()],
    out_specs=[_row_spec(), _row_spec()],
    out_shape=[jax.ShapeDtypeStruct((N, H), jnp.float32)] * 2,
)

_tc3 = pl.pallas_call(
    _tc3_body,
    grid=GRID,
    in_specs=[_row_spec(), _agg_spec(0), _agg_spec(1), _deg_spec(0),
              _deg_spec(1)],
    out_specs=_row_spec(),
    out_shape=jax.ShapeDtypeStruct((N, H), jnp.float32),
)


def kernel(x, edge_index, W_self0, W_neigh0, b0, W_self1, W_neigh1, b1):
    pad = E_PAD - E
    src3 = jnp.concatenate([edge_index[0], jnp.zeros((pad,), jnp.int32)])
    dst3 = jnp.concatenate([edge_index[1], jnp.full((pad,), N, jnp.int32)])
    w0 = jnp.concatenate([W_self0, W_neigh0], axis=1)
    w1 = jnp.concatenate([W_self1, W_neigh1], axis=1)

    s0, p0 = _tc1(x, w0, b0.reshape(1, H))
    agg0, deg0 = _sc_agg_deg(src3, dst3, p0)
    deg3 = deg0.reshape(NC, NP, 1)
    s1, p1 = _tc2(s0, agg0, agg0, deg3, deg3, w1, b1.reshape(1, H))
    agg1 = _sc_agg(src3, dst3, p1)
    out = _tc3(s1, agg1, agg1, deg3, deg3)
    return out


# speedup vs baseline: 1.1135x; 1.0026x over previous
import jax
import jax.numpy as jnp
from jax import lax
from jax.experimental import pallas as pl
from jax.experimental.pallas import tpu as pltpu
from jax.experimental.pallas import tpu_sc as plsc

N = 10000
E = 320000
D = 128
H = 128

NC = 2
NS = 16
NW = NC * NS

K = 128
C0 = 141
C1 = 17
EW0 = C0 * K
EW1 = C1 * K
E_PAD = NS * (EW0 + EW1)
RPS = 640
NP = NS * RPS


def _sc_aggregate(with_deg: bool):
    agg_t = jax.ShapeDtypeStruct((NC, NP, D), jnp.float32)
    if with_deg:
        out_type = [agg_t, jax.ShapeDtypeStruct((NC * NP,), jnp.float32)]
    else:
        out_type = agg_t

    scratch = [
        pltpu.VMEM_SHARED((NP, D), jnp.float32),
        pltpu.VMEM_SHARED((NP,), jnp.float32),
        pltpu.VMEM((2, K), jnp.int32),
        pltpu.VMEM((2, K), jnp.int32),
        pltpu.VMEM((K, D), jnp.float32),
        pltpu.VMEM((K, D), jnp.float32),
        pltpu.VMEM((K,), jnp.float32),
        pltpu.VMEM((RPS,), jnp.float32),
    ] + [pltpu.SemaphoreType.DMA] * 6

    def body(srcf, dstf, p, *rest):
        if with_deg:
            agg_out, deg_out = rest[0], rest[1]
            rest = rest[2:]
        else:
            agg_out, deg_out = rest[0], None
            rest = rest[1:]
        acc_sh, deg_sh, srcb, dstb = rest[0], rest[1], rest[2], rest[3]
        rows = rest[4:6]
        ones_v, deg_v = rest[6], rest[7]
        semr = rest[8:10]
        semis = rest[10:12]
        semid = rest[12:14]

        cid = lax.axis_index("c")
        sid = lax.axis_index("s")
        r0 = sid * RPS
        is0 = cid == 0
        base = jnp.where(is0, sid * EW0, NS * EW0 + sid * EW1)
        T = jnp.where(is0, (C0 - 3) // 2, (C1 - 3) // 2)

        def issue_idx(j, slot):
            off = base + j * K
            pltpu.async_copy(srcf.at[pl.ds(off, K)], srcb.at[slot],
                             semis[slot])
            pltpu.async_copy(dstf.at[pl.ds(off, K)], dstb.at[slot],
                             semid[slot])

        def wait_src(slot):
            pltpu.make_async_copy(
                srcf.at[pl.ds(base, K)], srcb.at[slot], semis[slot]).wait()

        def wait_dst(slot):
            pltpu.make_async_copy(
                dstf.at[pl.ds(base, K)], dstb.at[slot], semid[slot]).wait()

        def gather(slot):
            pltpu.async_copy(p.at[srcb.at[slot]], rows[slot], semr[slot])

        def wait_rows(slot):
            pltpu.make_async_copy(
                p.at[srcb.at[slot]], rows[slot], semr[slot]).wait()

        def scatter(slot):
            pltpu.sync_copy(rows[slot], acc_sh.at[dstb.at[slot]], add=True)
            if with_deg:
                pltpu.sync_copy(ones_v, deg_sh.at[dstb.at[slot]], add=True)

        def _work():
            def zrow(i, carry):
                for c in range(D // 16):
                    rows[0][i, pl.ds(c * 16, 16)] = jnp.zeros(
                        (16,), jnp.float32)
                return carry
            lax.fori_loop(0, K, zrow, 0)
            for t in range(RPS // K):
                pltpu.sync_copy(rows[0], acc_sh.at[pl.ds(r0 + t * K, K)])
            if with_deg:
                def zstep(i, carry):
                    deg_v[pl.ds(i * 16, 16)] = jnp.zeros((16,), jnp.float32)
                    return carry
                lax.fori_loop(0, RPS // 16, zstep, 0)
                pltpu.sync_copy(deg_v, deg_sh.at[pl.ds(r0, RPS)])
                for i in range(K // 16):
                    ones_v[pl.ds(i * 16, 16)] = jnp.ones((16,), jnp.float32)
            plsc.subcore_barrier()

            def half(jc, cur, nxt, issue, gather_next):
                if gather_next:
                    wait_src(nxt)
                    gather(nxt)
                wait_rows(cur)
                wait_dst(cur)
                scatter(cur)
                if issue:
                    issue_idx(jc + 2, cur)

            issue_idx(0, 0)
            issue_idx(1, 1)
            wait_src(0)
            gather(0)

            def step(i, carry):
                j = 2 * i
                half(j, 0, 1, True, True)
                half(j + 1, 1, 0, True, True)
                return carry

            lax.fori_loop(0, T, step, 0)
            half(2 * T, 0, 1, True, True)
            half(2 * T + 1, 1, 0, False, True)
            half(2 * T + 2, 0, 1, False, False)
            plsc.subcore_barrier()

            wfull = RPS // K
            pltpu.sync_copy(acc_sh.at[pl.ds(r0, K)], rows[0])
            pltpu.async_copy(rows[0], agg_out.at[cid, pl.ds(r0, K)], semr[0])
            for t in range(1, wfull):
                cur = rows[t % 2]
                pltpu.sync_copy(acc_sh.at[pl.ds(r0 + t * K, K)], cur)
                pltpu.async_copy(cur, agg_out.at[cid, pl.ds(r0 + t * K, K)],
                                 semr[t % 2])
                prev = rows[(t - 1) % 2]
                pltpu.make_async_copy(prev, agg_out.at[cid, pl.ds(r0, K)],
                                      semr[(t - 1) % 2]).wait()
            pltpu.make_async_copy(
                rows[(wfull - 1) % 2], agg_out.at[cid, pl.ds(r0, K)],
                semr[(wfull - 1) % 2]).wait()
            if with_deg:
                pltpu.sync_copy(deg_sh.at[pl.ds(r0, RPS)], deg_v)
                pltpu.sync_copy(deg_v,
                                deg_out.at[pl.ds(cid * NP + r0, RPS)])

        _work()

    return pl.kernel(
        body,
        out_type=out_type,
        mesh=plsc.VectorSubcoreMesh(core_axis_name="c", subcore_axis_name="s"),
        scratch_types=scratch,
    )


_sc_agg_deg = _sc_aggregate(with_deg=True)
_sc_agg = _sc_aggregate(with_deg=False)


BN = 1000
GRID = (N // BN,)


def _tc1_body(x_ref, w_ref, b_ref, s_ref, p_ref):
    y = jnp.dot(x_ref[...], w_ref[...], preferred_element_type=jnp.float32)
    s_ref[...] = y[:, :H] + b_ref[...]
    p_ref[...] = y[:, H:]


def _tc2_body(s0_ref, a0_ref, a1_ref, d0_ref, d1_ref, w_ref, b_ref,
              s_ref, p_ref):
    deg = jnp.squeeze(d0_ref[...] + d1_ref[...], axis=0)
    inv = 1.0 / jnp.maximum(deg, 1.0)
    agg = jnp.squeeze(a0_ref[...] + a1_ref[...], axis=0)
    h = jnp.maximum(s0_ref[...] + agg * inv, 0.0)
    y = jnp.dot(h, w_ref[...], preferred_element_type=jnp.float32)
    s_ref[...] = y[:, :H] + b_ref[...]
    p_ref[...] = y[:, H:]


def _tc3_body(s1_ref, a0_ref, a1_ref, d0_ref, d1_ref, o_ref):
    deg = jnp.squeeze(d0_ref[...] + d1_ref[...], axis=0)
    inv = 1.0 / jnp.maximum(deg, 1.0)
    agg = jnp.squeeze(a0_ref[...] + a1_ref[...], axis=0)
    o_ref[...] = s1_ref[...] + agg * inv


def _row_spec():
    return pl.BlockSpec((BN, D), lambda i: (i, 0))


def _w_spec():
    return pl.BlockSpec((D, 2 * H), lambda i: (0, 0))


def _b_spec():
    return pl.BlockSpec((1, H), lambda i: (0, 0))


def _agg_spec(c):
    return pl.BlockSpec((1, BN, H), lambda i, c=c: (c, i, 0))


def _deg_spec(c):
    return pl.BlockSpec((1, BN, 1), lambda i, c=c: (c, i, 0))


_tc1 = pl.pallas_call(
    _tc1_body,
    grid=GRID,
    in_specs=[_row_spec(), _w_spec(), _b_spec()],
    out_specs=[_row_spec(), _row_spec()],
    out_shape=[jax.ShapeDtypeStruct((N, H), jnp.float32)] * 2,
)

_tc2 = pl.pallas_call(
    _tc2_body,
    grid=GRID,
    in_specs=[_row_spec(), _agg_spec(0), _agg_spec(1), _deg_spec(0),
              _deg_spec(1), _w_spec(), _b_spec()],
    out_specs=[_row_spec(), _row_spec()],
    out_shape=[jax.ShapeDtypeStruct((N, H), jnp.float32)] * 2,
)

_tc3 = pl.pallas_call(
    _tc3_body,
    grid=GRID,
    in_specs=[_row_spec(), _agg_spec(0), _agg_spec(1), _deg_spec(0),
              _deg_spec(1)],
    out_specs=_row_spec(),
    out_shape=jax.ShapeDtypeStruct((N, H), jnp.float32),
)


def kernel(x, edge_index, W_self0, W_neigh0, b0, W_self1, W_neigh1, b1):
    pad = E_PAD - E
    src3 = jnp.concatenate([edge_index[0], jnp.zeros((pad,), jnp.int32)])
    dst3 = jnp.concatenate([edge_index[1], jnp.full((pad,), N, jnp.int32)])
    w0 = jnp.concatenate([W_self0, W_neigh0], axis=1)
    w1 = jnp.concatenate([W_self1, W_neigh1], axis=1)

    s0, p0 = _tc1(x, w0, b0.reshape(1, H))
    agg0, deg0 = _sc_agg_deg(src3, dst3, p0)
    deg3 = deg0.reshape(NC, NP, 1)
    s1, p1 = _tc2(s0, agg0, agg0, deg3, deg3, w1, b1.reshape(1, H))
    agg1 = _sc_agg(src3, dst3, p1)
    out = _tc3(s1, agg1, agg1, deg3, deg3)
    return out


# speedup vs baseline: 1.1207x; 1.0065x over previous
import jax
import jax.numpy as jnp
from jax import lax
from jax.experimental import pallas as pl
from jax.experimental.pallas import tpu as pltpu
from jax.experimental.pallas import tpu_sc as plsc

N = 10000
E = 320000
D = 128
H = 128

NC = 2
NS = 16
NW = NC * NS

K = 128
C0 = 143
C1 = 15
EW0 = C0 * K
EW1 = C1 * K
E_PAD = NS * (EW0 + EW1)
RPS = 640
NP = NS * RPS


def _sc_aggregate(with_deg: bool):
    agg_t = jax.ShapeDtypeStruct((NC, NP, D), jnp.float32)
    if with_deg:
        out_type = [agg_t, jax.ShapeDtypeStruct((NC * NP,), jnp.float32)]
    else:
        out_type = agg_t

    scratch = [
        pltpu.VMEM_SHARED((NP, D), jnp.float32),
        pltpu.VMEM_SHARED((NP,), jnp.float32),
        pltpu.VMEM((2, K), jnp.int32),
        pltpu.VMEM((2, K), jnp.int32),
        pltpu.VMEM((K, D), jnp.float32),
        pltpu.VMEM((K, D), jnp.float32),
        pltpu.VMEM((K,), jnp.float32),
        pltpu.VMEM((RPS,), jnp.float32),
    ] + [pltpu.SemaphoreType.DMA] * 6

    def body(srcf, dstf, p, *rest):
        if with_deg:
            agg_out, deg_out = rest[0], rest[1]
            rest = rest[2:]
        else:
            agg_out, deg_out = rest[0], None
            rest = rest[1:]
        acc_sh, deg_sh, srcb, dstb = rest[0], rest[1], rest[2], rest[3]
        rows = rest[4:6]
        ones_v, deg_v = rest[6], rest[7]
        semr = rest[8:10]
        semis = rest[10:12]
        semid = rest[12:14]

        cid = lax.axis_index("c")
        sid = lax.axis_index("s")
        r0 = sid * RPS
        is0 = cid == 0
        base = jnp.where(is0, sid * EW0, NS * EW0 + sid * EW1)
        T = jnp.where(is0, (C0 - 3) // 2, (C1 - 3) // 2)

        def issue_idx(j, slot):
            off = base + j * K
            pltpu.async_copy(srcf.at[pl.ds(off, K)], srcb.at[slot],
                             semis[slot])
            pltpu.async_copy(dstf.at[pl.ds(off, K)], dstb.at[slot],
                             semid[slot])

        def wait_src(slot):
            pltpu.make_async_copy(
                srcf.at[pl.ds(base, K)], srcb.at[slot], semis[slot]).wait()

        def wait_dst(slot):
            pltpu.make_async_copy(
                dstf.at[pl.ds(base, K)], dstb.at[slot], semid[slot]).wait()

        def gather(slot):
            pltpu.async_copy(p.at[srcb.at[slot]], rows[slot], semr[slot])

        def wait_rows(slot):
            pltpu.make_async_copy(
                p.at[srcb.at[slot]], rows[slot], semr[slot]).wait()

        def scatter(slot):
            pltpu.sync_copy(rows[slot], acc_sh.at[dstb.at[slot]], add=True)
            if with_deg:
                pltpu.sync_copy(ones_v, deg_sh.at[dstb.at[slot]], add=True)

        def _work():
            def zrow(i, carry):
                for c in range(D // 16):
                    rows[0][i, pl.ds(c * 16, 16)] = jnp.zeros(
                        (16,), jnp.float32)
                return carry
            lax.fori_loop(0, K, zrow, 0)
            for t in range(RPS // K):
                pltpu.sync_copy(rows[0], acc_sh.at[pl.ds(r0 + t * K, K)])
            if with_deg:
                def zstep(i, carry):
                    deg_v[pl.ds(i * 16, 16)] = jnp.zeros((16,), jnp.float32)
                    return carry
                lax.fori_loop(0, RPS // 16, zstep, 0)
                pltpu.sync_copy(deg_v, deg_sh.at[pl.ds(r0, RPS)])
                for i in range(K // 16):
                    ones_v[pl.ds(i * 16, 16)] = jnp.ones((16,), jnp.float32)
            plsc.subcore_barrier()

            def half(jc, cur, nxt, issue, gather_next):
                if gather_next:
                    wait_src(nxt)
                    gather(nxt)
                wait_rows(cur)
                wait_dst(cur)
                scatter(cur)
                if issue:
                    issue_idx(jc + 2, cur)

            issue_idx(0, 0)
            issue_idx(1, 1)
            wait_src(0)
            gather(0)

            def step(i, carry):
                j = 2 * i
                half(j, 0, 1, True, True)
                half(j + 1, 1, 0, True, True)
                return carry

            lax.fori_loop(0, T, step, 0)
            half(2 * T, 0, 1, True, True)
            half(2 * T + 1, 1, 0, False, True)
            half(2 * T + 2, 0, 1, False, False)
            plsc.subcore_barrier()

            wfull = RPS // K
            pltpu.sync_copy(acc_sh.at[pl.ds(r0, K)], rows[0])
            pltpu.async_copy(rows[0], agg_out.at[cid, pl.ds(r0, K)], semr[0])
            for t in range(1, wfull):
                cur = rows[t % 2]
                pltpu.sync_copy(acc_sh.at[pl.ds(r0 + t * K, K)], cur)
                pltpu.async_copy(cur, agg_out.at[cid, pl.ds(r0 + t * K, K)],
                                 semr[t % 2])
                prev = rows[(t - 1) % 2]
                pltpu.make_async_copy(prev, agg_out.at[cid, pl.ds(r0, K)],
                                      semr[(t - 1) % 2]).wait()
            pltpu.make_async_copy(
                rows[(wfull - 1) % 2], agg_out.at[cid, pl.ds(r0, K)],
                semr[(wfull - 1) % 2]).wait()
            if with_deg:
                pltpu.sync_copy(deg_sh.at[pl.ds(r0, RPS)], deg_v)
                pltpu.sync_copy(deg_v,
                                deg_out.at[pl.ds(cid * NP + r0, RPS)])

        _work()

    return pl.kernel(
        body,
        out_type=out_type,
        mesh=plsc.VectorSubcoreMesh(core_axis_name="c", subcore_axis_name="s"),
        scratch_types=scratch,
    )


_sc_agg_deg = _sc_aggregate(with_deg=True)
_sc_agg = _sc_aggregate(with_deg=False)


BN = 1000
GRID = (N // BN,)


def _tc1_body(x_ref, w_ref, b_ref, s_ref, p_ref):
    y = jnp.dot(x_ref[...], w_ref[...], preferred_element_type=jnp.float32)
    s_ref[...] = y[:, :H] + b_ref[...]
    p_ref[...] = y[:, H:]


def _tc2_body(s0_ref, a0_ref, a1_ref, d0_ref, d1_ref, w_ref, b_ref,
              s_ref, p_ref):
    deg = jnp.squeeze(d0_ref[...] + d1_ref[...], axis=0)
    inv = 1.0 / jnp.maximum(deg, 1.0)
    agg = jnp.squeeze(a0_ref[...] + a1_ref[...], axis=0)
    h = jnp.maximum(s0_ref[...] + agg * inv, 0.0)
    y = jnp.dot(h, w_ref[...], preferred_element_type=jnp.float32)
    s_ref[...] = y[:, :H] + b_ref[...]
    p_ref[...] = y[:, H:]


def _tc3_body(s1_ref, a0_ref, a1_ref, d0_ref, d1_ref, o_ref):
    deg = jnp.squeeze(d0_ref[...] + d1_ref[...], axis=0)
    inv = 1.0 / jnp.maximum(deg, 1.0)
    agg = jnp.squeeze(a0_ref[...] + a1_ref[...], axis=0)
    o_ref[...] = s1_ref[...] + agg * inv


def _row_spec():
    return pl.BlockSpec((BN, D), lambda i: (i, 0))


def _w_spec():
    return pl.BlockSpec((D, 2 * H), lambda i: (0, 0))


def _b_spec():
    return pl.BlockSpec((1, H), lambda i: (0, 0))


def _agg_spec(c):
    return pl.BlockSpec((1, BN, H), lambda i, c=c: (c, i, 0))


def _deg_spec(c):
    return pl.BlockSpec((1, BN, 1), lambda i, c=c: (c, i, 0))


_tc1 = pl.pallas_call(
    _tc1_body,
    grid=GRID,
    in_specs=[_row_spec(), _w_spec(), _b_spec()],
    out_specs=[_row_spec(), _row_spec()],
    out_shape=[jax.ShapeDtypeStruct((N, H), jnp.float32)] * 2,
)

_tc2 = pl.pallas_call(
    _tc2_body,
    grid=GRID,
    in_specs=[_row_spec(), _agg_spec(0), _agg_spec(1), _deg_spec(0),
              _deg_spec(1), _w_spec(), _b_spec()],
    out_specs=[_row_spec(), _row_spec()],
    out_shape=[jax.ShapeDtypeStruct((N, H), jnp.float32)] * 2,
)

_tc3 = pl.pallas_call(
    _tc3_body,
    grid=GRID,
    in_specs=[_row_spec(), _agg_spec(0), _agg_spec(1), _deg_spec(0),
              _deg_spec(1)],
    out_specs=_row_spec(),
    out_shape=jax.ShapeDtypeStruct((N, H), jnp.float32),
)


def kernel(x, edge_index, W_self0, W_neigh0, b0, W_self1, W_neigh1, b1):
    pad = E_PAD - E
    src3 = jnp.concatenate([edge_index[0], jnp.zeros((pad,), jnp.int32)])
    dst3 = jnp.concatenate([edge_index[1], jnp.full((pad,), N, jnp.int32)])
    w0 = jnp.concatenate([W_self0, W_neigh0], axis=1)
    w1 = jnp.concatenate([W_self1, W_neigh1], axis=1)

    s0, p0 = _tc1(x, w0, b0.reshape(1, H))
    agg0, deg0 = _sc_agg_deg(src3, dst3, p0)
    deg3 = deg0.reshape(NC, NP, 1)
    s1, p1 = _tc2(s0, agg0, agg0, deg3, deg3, w1, b1.reshape(1, H))
    agg1 = _sc_agg(src3, dst3, p1)
    out = _tc3(s1, agg1, agg1, deg3, deg3)
    return out
